# Initial kernel scaffold; baseline (speedup 1.0000x reference)
import jax
import jax.numpy as jnp
from jax.experimental import pallas as pl


def kernel(x, k):
    raise NotImplementedError("write your pallas kernel here")


# speedup vs baseline: 3.0716x; 3.0716x over previous
import functools

import jax
import jax.numpy as jnp
from jax import lax
from jax.experimental import pallas as pl

_K = 20


def _edge_body(x_all_ref, x_rows_ref, xx_ref, xxt_ref, o_ref, *, R, K):
    xa = x_all_ref[0]
    xr = x_rows_ref[0]
    d, N = xa.shape

    inner2 = lax.dot_general(xr, xa, (((0,), (0,)), ((), ())),
                             preferred_element_type=jnp.float32)
    inner = -2.0 * inner2
    pd = (-xx_ref[0]) - inner
    pd = pd - xxt_ref[0]

    iota_m = lax.broadcasted_iota(jnp.int32, (R, N), 1)
    neg_inf = jnp.float32(-jnp.inf)

    xab = xa.astype(jnp.bfloat16)

    oh_parts = []
    pdw = pd
    for j in range(K):
        am = jnp.argmax(pdw, axis=1).reshape(R, 1)
        hit = iota_m == am
        oh_parts.append(hit.astype(jnp.bfloat16).reshape(R, 1, N))
        if j < K - 1:
            pdw = jnp.where(hit, neg_inf, pdw)

    oh2 = jnp.concatenate(oh_parts, axis=1).reshape(R * K, N)

    F = lax.dot_general(oh2, xab, (((1,), (1,)), ((), ())),
                        preferred_element_type=jnp.float32)

    eye = (lax.broadcasted_iota(jnp.int32, (d, d), 0)
           == lax.broadcasted_iota(jnp.int32, (d, d), 1)).astype(jnp.bfloat16)
    Ft = lax.dot_general(eye, F.astype(jnp.bfloat16), (((1,), (1,)), ((), ())),
                         preferred_element_type=jnp.float32)

    li = lax.broadcasted_iota(jnp.int32, (R, R * K), 1)
    si = lax.broadcasted_iota(jnp.int32, (R, R * K), 0)
    rep = ((li >= si * K) & (li < si * K + K)).astype(jnp.float32)
    Ct = lax.dot_general(xr, rep, (((1,), (0,)), ((), ())),
                         preferred_element_type=jnp.float32)

    o_ref[0, 0:d, :] = Ft - Ct
    o_ref[0, d:2 * d, :] = Ct


def kernel(x, k):
    B, d, N = x.shape
    K = _K
    R = 128

    xx = jnp.sum(x * x, axis=1, keepdims=True)
    xxt = jnp.swapaxes(xx, 1, 2)

    body = functools.partial(_edge_body, R=R, K=K)
    out_flat = pl.pallas_call(
        body,
        grid=(B, N // R),
        in_specs=[
            pl.BlockSpec((1, d, N), lambda b, i: (b, 0, 0)),
            pl.BlockSpec((1, d, R), lambda b, i: (b, 0, i)),
            pl.BlockSpec((1, 1, N), lambda b, i: (b, 0, 0)),
            pl.BlockSpec((1, R, 1), lambda b, i: (b, i, 0)),
        ],
        out_specs=pl.BlockSpec((1, 2 * d, R * K), lambda b, i: (b, 0, i)),
        out_shape=jax.ShapeDtypeStruct((B, 2 * d, N * K), jnp.float32),
    )(x, x, xx, xxt)
    return out_flat.reshape(B, 2 * d, N, K)


# speedup vs baseline: 5.5169x; 1.7961x over previous
import functools

import jax
import jax.numpy as jnp
from jax import lax
from jax.experimental import pallas as pl

_K = 20


def _edge_body(x_all_ref, x_rows_ref, xx_ref, xxt_ref, o_ref, *, R, K):
    xa = x_all_ref[0]
    xr = x_rows_ref[0]
    d, N = xa.shape

    inner2 = lax.dot_general(xr, xa, (((0,), (0,)), ((), ())),
                             preferred_element_type=jnp.float32)
    inner = -2.0 * inner2
    pd = (-xx_ref[0]) - inner
    pd = pd - xxt_ref[0]

    iota_m = lax.broadcasted_iota(jnp.int32, (R, N), 1)
    neg_inf = jnp.float32(-jnp.inf)

    xab = xa.astype(jnp.bfloat16)

    idx_parts = []
    pdw = pd
    for j in range(K):
        am = jnp.argmax(pdw, axis=1, keepdims=True)
        idx_parts.append(am.reshape(R, 1, 1))
        if j < K - 1:
            pdw = jnp.where(iota_m == am, neg_inf, pdw)

    idx3 = jnp.concatenate(idx_parts, axis=1)
    iota3 = lax.broadcasted_iota(jnp.int32, (R, K, N), 2)
    oh2 = (idx3 == iota3).astype(jnp.bfloat16).reshape(R * K, N)

    F = lax.dot_general(oh2, xab, (((1,), (1,)), ((), ())),
                        preferred_element_type=jnp.float32)

    eye = (lax.broadcasted_iota(jnp.int32, (d, d), 0)
           == lax.broadcasted_iota(jnp.int32, (d, d), 1)).astype(jnp.bfloat16)
    Ft = lax.dot_general(eye, F.astype(jnp.bfloat16), (((1,), (1,)), ((), ())),
                         preferred_element_type=jnp.float32)

    li = lax.broadcasted_iota(jnp.int32, (R, R * K), 1)
    si = lax.broadcasted_iota(jnp.int32, (R, R * K), 0)
    rep = ((li >= si * K) & (li < si * K + K)).astype(jnp.float32)
    Ct = lax.dot_general(xr, rep, (((1,), (0,)), ((), ())),
                         preferred_element_type=jnp.float32)

    o_ref[0, 0:d, :] = Ft - Ct
    o_ref[0, d:2 * d, :] = Ct


def kernel(x, k):
    B, d, N = x.shape
    K = _K
    R = 128

    xx = jnp.sum(x * x, axis=1, keepdims=True)
    xxt = jnp.swapaxes(xx, 1, 2)

    body = functools.partial(_edge_body, R=R, K=K)
    out_flat = pl.pallas_call(
        body,
        grid=(B, N // R),
        in_specs=[
            pl.BlockSpec((1, d, N), lambda b, i: (b, 0, 0)),
            pl.BlockSpec((1, d, R), lambda b, i: (b, 0, i)),
            pl.BlockSpec((1, 1, N), lambda b, i: (b, 0, 0)),
            pl.BlockSpec((1, R, 1), lambda b, i: (b, i, 0)),
        ],
        out_specs=pl.BlockSpec((1, 2 * d, R * K), lambda b, i: (b, 0, i)),
        out_shape=jax.ShapeDtypeStruct((B, 2 * d, N * K), jnp.float32),
    )(x, x, xx, xxt)
    return out_flat.reshape(B, 2 * d, N, K)


# speedup vs baseline: 5.7754x; 1.0468x over previous
import functools

import jax
import jax.numpy as jnp
from jax import lax
from jax.experimental import pallas as pl

_K = 20


def _edge_body(x_all_ref, x_rows_ref, xx_ref, xxt_ref, o_ref, *, R, K):
    xa = x_all_ref[0]
    xr = x_rows_ref[0]
    d, N = xa.shape

    inner2 = lax.dot_general(xr, xa, (((0,), (0,)), ((), ())),
                             preferred_element_type=jnp.float32)
    inner = -2.0 * inner2
    pd = (-xx_ref[0]) - inner
    pd = pd - xxt_ref[0]

    iota_m = lax.broadcasted_iota(jnp.int32, (R, N), 1)
    neg_inf = jnp.float32(-jnp.inf)

    xab = xa.astype(jnp.bfloat16)

    idx_parts = []
    pdw = pd
    for j in range(K):
        am = jnp.argmax(pdw, axis=1, keepdims=True)
        idx_parts.append(am.reshape(R, 1, 1))
        if j < K - 1:
            pdw = jnp.where(iota_m == am, neg_inf, pdw)

    idx3 = jnp.concatenate(idx_parts, axis=1)
    iota3 = lax.broadcasted_iota(jnp.int32, (R, K, N), 2)
    oh2 = (idx3 == iota3).astype(jnp.bfloat16).reshape(R * K, N)

    F = lax.dot_general(oh2, xab, (((1,), (1,)), ((), ())),
                        preferred_element_type=jnp.float32)

    eye = (lax.broadcasted_iota(jnp.int32, (d, d), 0)
           == lax.broadcasted_iota(jnp.int32, (d, d), 1)).astype(jnp.bfloat16)
    Ft = lax.dot_general(eye, F.astype(jnp.bfloat16), (((1,), (1,)), ((), ())),
                         preferred_element_type=jnp.float32)

    li = lax.broadcasted_iota(jnp.int32, (R, R * K), 1)
    si = lax.broadcasted_iota(jnp.int32, (R, R * K), 0)
    rep = ((li >= si * K) & (li < si * K + K)).astype(jnp.float32)
    Ct = lax.dot_general(xr, rep, (((1,), (0,)), ((), ())),
                         preferred_element_type=jnp.float32)

    o_ref[0, 0:d, :] = Ft - Ct
    o_ref[0, d:2 * d, :] = Ct


def kernel(x, k):
    B, d, N = x.shape
    K = _K
    R = 256

    xx = jnp.sum(x * x, axis=1, keepdims=True)
    xxt = jnp.swapaxes(xx, 1, 2)

    body = functools.partial(_edge_body, R=R, K=K)
    out_flat = pl.pallas_call(
        body,
        grid=(B, N // R),
        in_specs=[
            pl.BlockSpec((1, d, N), lambda b, i: (b, 0, 0)),
            pl.BlockSpec((1, d, R), lambda b, i: (b, 0, i)),
            pl.BlockSpec((1, 1, N), lambda b, i: (b, 0, 0)),
            pl.BlockSpec((1, R, 1), lambda b, i: (b, i, 0)),
        ],
        out_specs=pl.BlockSpec((1, 2 * d, R * K), lambda b, i: (b, 0, i)),
        out_shape=jax.ShapeDtypeStruct((B, 2 * d, N * K), jnp.float32),
    )(x, x, xx, xxt)
    return out_flat.reshape(B, 2 * d, N, K)


# speedup vs baseline: 6.6362x; 1.1490x over previous
import functools

import jax
import jax.numpy as jnp
from jax import lax
from jax.experimental import pallas as pl
from jax.experimental.pallas import tpu as pltpu
from jax.experimental.pallas import tpu_sc as plsc

_K = 20


def _topk_body(x_all_ref, x_rows_ref, xx_ref, xxt_ref, idx_ref, xt_ref, *, R, K):
    xa = x_all_ref[0]
    xr = x_rows_ref[0]
    d, N = xa.shape
    b = pl.program_id(0)

    inner2 = lax.dot_general(xr, xa, (((0,), (0,)), ((), ())),
                             preferred_element_type=jnp.float32)
    inner = -2.0 * inner2
    pd = (-xx_ref[0]) - inner
    pd = pd - xxt_ref[0]

    iota_m = lax.broadcasted_iota(jnp.int32, (R, N), 1)
    neg_inf = jnp.float32(-jnp.inf)

    pdw = pd
    for j in range(K):
        am = jnp.argmax(pdw, axis=1, keepdims=True)
        idx_ref[0, :, j:j + 1] = am + b * N
        if j < K - 1:
            pdw = jnp.where(iota_m == am, neg_inf, pdw)

    eyeR = (lax.broadcasted_iota(jnp.int32, (R, R), 0)
            == lax.broadcasted_iota(jnp.int32, (R, R), 1)).astype(jnp.float32)
    xt_ref[0] = lax.dot_general(eyeR, xr, (((1,), (1,)), ((), ())),
                                preferred_element_type=jnp.float32)


def _assemble_body(g_ref, x_rows_ref, o_ref, *, R, K):
    xr = x_rows_ref[0]
    d = xr.shape[0]
    Gb = g_ref[0]

    eye = (lax.broadcasted_iota(jnp.int32, (d, d), 0)
           == lax.broadcasted_iota(jnp.int32, (d, d), 1)).astype(jnp.float32)
    Ft = lax.dot_general(eye, Gb, (((1,), (1,)), ((), ())),
                         preferred_element_type=jnp.float32)

    li = lax.broadcasted_iota(jnp.int32, (R, R * K), 1)
    si = lax.broadcasted_iota(jnp.int32, (R, R * K), 0)
    rep = ((li >= si * K) & (li < si * K + K)).astype(jnp.float32)
    Ct = lax.dot_general(xr, rep, (((1,), (0,)), ((), ())),
                         preferred_element_type=jnp.float32)

    o_ref[0, 0:d, :] = Ft - Ct
    o_ref[0, d:2 * d, :] = Ct


def _sc_gather(BNK, d, CH, per_w, NC):
    mesh = plsc.VectorSubcoreMesh(core_axis_name="c", subcore_axis_name="s")

    @functools.partial(
        pl.kernel, mesh=mesh,
        out_type=jax.ShapeDtypeStruct((BNK, d), jnp.float32),
        scratch_types=[
            pltpu.VMEM((CH,), jnp.int32),
            pltpu.VMEM((CH, d), jnp.float32),
            pltpu.SemaphoreType.DMA,
        ],
    )
    def gathr(xt_hbm, gidx_hbm, out_hbm, idx_v, rows_v, sem):
        wid = lax.axis_index("s") * NC + lax.axis_index("c")
        base = wid * per_w

        def body(t, carry):
            off = base + t * CH
            pltpu.sync_copy(gidx_hbm.at[pl.ds(off, CH)], idx_v)
            pltpu.async_copy(xt_hbm.at[idx_v], rows_v, sem).wait()
            pltpu.sync_copy(rows_v, out_hbm.at[pl.ds(off, CH)])
            return carry

        lax.fori_loop(0, per_w // CH, body, 0)

    return gathr


def kernel(x, k):
    B, d, N = x.shape
    K = _K
    R = 256

    xx = jnp.sum(x * x, axis=1, keepdims=True)
    xxt = jnp.swapaxes(xx, 1, 2)

    topk = functools.partial(_topk_body, R=R, K=K)
    gidx, xt = pl.pallas_call(
        topk,
        grid=(B, N // R),
        in_specs=[
            pl.BlockSpec((1, d, N), lambda b, i: (b, 0, 0)),
            pl.BlockSpec((1, d, R), lambda b, i: (b, 0, i)),
            pl.BlockSpec((1, 1, N), lambda b, i: (b, 0, 0)),
            pl.BlockSpec((1, R, 1), lambda b, i: (b, i, 0)),
        ],
        out_specs=[
            pl.BlockSpec((1, R, K), lambda b, i: (b, i, 0)),
            pl.BlockSpec((1, R, d), lambda b, i: (b, i, 0)),
        ],
        out_shape=[
            jax.ShapeDtypeStruct((B, N, K), jnp.int32),
            jax.ShapeDtypeStruct((B, N, d), jnp.float32),
        ],
    )(x, x, xx, xxt)

    info = plsc.get_sparse_core_info()
    NC, NS = info.num_cores, info.num_subcores
    NW = NC * NS
    BNK = B * N * K
    per_w = BNK // NW
    CH = 128

    G = _sc_gather(BNK, d, CH, per_w, NC)(
        xt.reshape(B * N, d), gidx.reshape(BNK))

    asm = functools.partial(_assemble_body, R=R, K=K)
    out_flat = pl.pallas_call(
        asm,
        grid=(B, N // R),
        in_specs=[
            pl.BlockSpec((1, R * K, d), lambda b, i: (b, i, 0)),
            pl.BlockSpec((1, d, R), lambda b, i: (b, 0, i)),
        ],
        out_specs=pl.BlockSpec((1, 2 * d, R * K), lambda b, i: (b, 0, i)),
        out_shape=jax.ShapeDtypeStruct((B, 2 * d, N * K), jnp.float32),
    )(G.reshape(B, N * K, d), x)
    return out_flat.reshape(B, 2 * d, N, K)


# speedup vs baseline: 6.6582x; 1.0033x over previous
import functools

import jax
import jax.numpy as jnp
from jax import lax
from jax.experimental import pallas as pl
from jax.experimental.pallas import tpu as pltpu
from jax.experimental.pallas import tpu_sc as plsc

_K = 20


def _topk_body(x_all_ref, x_rows_ref, xx_ref, xxt_ref, idx_ref, xt_ref, *, R, K):
    xa = x_all_ref[0]
    xr = x_rows_ref[0]
    d, N = xa.shape
    b = pl.program_id(0)

    inner2 = lax.dot_general(xr, xa, (((0,), (0,)), ((), ())),
                             preferred_element_type=jnp.float32)
    inner = -2.0 * inner2
    pd = (-xx_ref[0]) - inner
    pd = pd - xxt_ref[0]

    iota_m = lax.broadcasted_iota(jnp.int32, (R, N), 1)
    neg_inf = jnp.float32(-jnp.inf)

    pdw = pd
    for j in range(K):
        am = jnp.argmax(pdw, axis=1, keepdims=True)
        idx_ref[0, :, j:j + 1] = am + b * N
        if j < K - 1:
            pdw = jnp.where(iota_m == am, neg_inf, pdw)

    eyeR = (lax.broadcasted_iota(jnp.int32, (R, R), 0)
            == lax.broadcasted_iota(jnp.int32, (R, R), 1)).astype(jnp.float32)
    xt_ref[0] = lax.dot_general(eyeR, xr, (((1,), (1,)), ((), ())),
                                preferred_element_type=jnp.float32)


def _assemble_body(g_ref, x_rows_ref, o_ref, *, R, K):
    xr = x_rows_ref[0]
    d = xr.shape[0]
    Gb = g_ref[0]

    eye = (lax.broadcasted_iota(jnp.int32, (d, d), 0)
           == lax.broadcasted_iota(jnp.int32, (d, d), 1)).astype(jnp.float32)
    Ft = lax.dot_general(eye, Gb, (((1,), (1,)), ((), ())),
                         preferred_element_type=jnp.float32)

    li = lax.broadcasted_iota(jnp.int32, (R, R * K), 1)
    si = lax.broadcasted_iota(jnp.int32, (R, R * K), 0)
    rep = ((li >= si * K) & (li < si * K + K)).astype(jnp.float32)
    Ct = lax.dot_general(xr, rep, (((1,), (0,)), ((), ())),
                         preferred_element_type=jnp.float32)

    o_ref[0, 0:d, :] = Ft - Ct
    o_ref[0, d:2 * d, :] = Ct


def _sc_gather(BNK, d, CH, per_w, NC):
    mesh = plsc.VectorSubcoreMesh(core_axis_name="c", subcore_axis_name="s")

    GF = 4

    @functools.partial(
        pl.kernel, mesh=mesh,
        out_type=jax.ShapeDtypeStruct((BNK, d), jnp.float32),
        scratch_types=[
            pltpu.VMEM((per_w,), jnp.int32),
            pltpu.VMEM((GF * CH, d), jnp.float32),
            pltpu.SemaphoreType.DMA,
            pltpu.SemaphoreType.DMA,
        ],
    )
    def gathr(xt_hbm, gidx_hbm, out_hbm, idx_v, rows_v, gsem, wsem):
        wid = lax.axis_index("s") * NC + lax.axis_index("c")
        base = wid * per_w
        pltpu.sync_copy(gidx_hbm.at[pl.ds(base, per_w)], idx_v)

        def body(t, carry):
            off = t * (GF * CH)
            for g in range(GF):
                pltpu.async_copy(
                    xt_hbm.at[idx_v.at[pl.ds(off + g * CH, CH)]],
                    rows_v.at[pl.ds(g * CH, CH)], gsem)
            for g in range(GF):
                pltpu.make_async_copy(
                    xt_hbm.at[idx_v.at[pl.ds(off + g * CH, CH)]],
                    rows_v.at[pl.ds(g * CH, CH)], gsem).wait()
            pltpu.async_copy(rows_v, out_hbm.at[pl.ds(base + off, GF * CH)],
                             wsem).wait()
            return carry

        lax.fori_loop(0, per_w // (GF * CH), body, 0)

    return gathr


def kernel(x, k):
    B, d, N = x.shape
    K = _K
    R = 256

    xx = jnp.sum(x * x, axis=1, keepdims=True)
    xxt = jnp.swapaxes(xx, 1, 2)

    topk = functools.partial(_topk_body, R=R, K=K)
    gidx, xt = pl.pallas_call(
        topk,
        grid=(B, N // R),
        in_specs=[
            pl.BlockSpec((1, d, N), lambda b, i: (b, 0, 0)),
            pl.BlockSpec((1, d, R), lambda b, i: (b, 0, i)),
            pl.BlockSpec((1, 1, N), lambda b, i: (b, 0, 0)),
            pl.BlockSpec((1, R, 1), lambda b, i: (b, i, 0)),
        ],
        out_specs=[
            pl.BlockSpec((1, R, K), lambda b, i: (b, i, 0)),
            pl.BlockSpec((1, R, d), lambda b, i: (b, i, 0)),
        ],
        out_shape=[
            jax.ShapeDtypeStruct((B, N, K), jnp.int32),
            jax.ShapeDtypeStruct((B, N, d), jnp.float32),
        ],
    )(x, x, xx, xxt)

    info = plsc.get_sparse_core_info()
    NC, NS = info.num_cores, info.num_subcores
    NW = NC * NS
    BNK = B * N * K
    per_w = BNK // NW
    CH = 128

    G = _sc_gather(BNK, d, CH, per_w, NC)(
        xt.reshape(B * N, d), gidx.reshape(BNK))

    asm = functools.partial(_assemble_body, R=R, K=K)
    out_flat = pl.pallas_call(
        asm,
        grid=(B, N // R),
        in_specs=[
            pl.BlockSpec((1, R * K, d), lambda b, i: (b, i, 0)),
            pl.BlockSpec((1, d, R), lambda b, i: (b, 0, i)),
        ],
        out_specs=pl.BlockSpec((1, 2 * d, R * K), lambda b, i: (b, 0, i)),
        out_shape=jax.ShapeDtypeStruct((B, 2 * d, N * K), jnp.float32),
    )(G.reshape(B, N * K, d), x)
    return out_flat.reshape(B, 2 * d, N, K)
